# baseline (speedup 1.0000x reference)
import functools

import jax
import jax.numpy as jnp
from jax import lax
from jax.experimental import pallas as pl
from jax.experimental.pallas import tpu as pltpu
from jax.experimental.pallas import tpu_sc as plsc

N = 10000
NPAD = 10240
E = 320000
D = 128
G = 64
NC = 2
NS = 16
L = 16
NW = NC * NS
EPW = E // NW
ECH = 80
NCHUNK = EPW // ECH
NPW = NPAD // NW
CN = G * NPAD

_mesh = plsc.VectorSubcoreMesh(core_axis_name="c", subcore_axis_name="s")


def _zero_1d(ref, n):
    z = jnp.zeros((L,), ref.dtype)

    def body(i, _):
        ref[pl.ds(i * L, L)] = z
        return 0

    lax.fori_loop(0, n // L, body, 0)


@functools.partial(
    pl.kernel,
    out_type=(
        jax.ShapeDtypeStruct((NC, NPAD), jnp.float32),
        jax.ShapeDtypeStruct((NC, G), jnp.float32),
    ),
    mesh=_mesh,
    scratch_types=(
        pltpu.VMEM_SHARED((NPAD,), jnp.float32),
        pltpu.VMEM_SHARED((G,), jnp.float32),
        pltpu.VMEM((NCHUNK, ECH), jnp.int32),
        pltpu.VMEM((ECH,), jnp.float32),
        pltpu.VMEM((ECH,), jnp.int32),
        pltpu.VMEM((ECH,), jnp.float32),
        pltpu.VMEM((NPAD // NS,), jnp.float32),
        pltpu.SemaphoreType.DMA,
    ),
)
def _sc_hist(dst_hbm, batch_hbm, deg_out, cnt_out,
             dacc, cacc, dblk, onev, bidx, bval, zbuf, sem):
    c = lax.axis_index("c")
    s = lax.axis_index("s")
    wid = s * NC + c

    dcp = pltpu.async_copy(dst_hbm.at[wid], dblk, sem)

    _zero_1d(zbuf, NPAD // NS)
    pltpu.sync_copy(zbuf.at[pl.ds(0, NPAD // NS)], dacc.at[pl.ds(s * (NPAD // NS), NPAD // NS)])

    @pl.when(s == 0)
    def _():
        pltpu.sync_copy(zbuf.at[pl.ds(0, G)], cacc)

    one = jnp.ones((L,), jnp.float32)
    for k in range(ECH // L):
        onev[pl.ds(k * L, L)] = one

    dcp.wait()
    plsc.subcore_barrier()

    def deg_body(j, _):
        pltpu.sync_copy(onev, dacc.at[dblk.at[j]], add=True)
        return 0

    lax.fori_loop(0, NCHUNK, deg_body, 0)

    nb = wid * NPW
    for q in range(NPW // ECH):
        pltpu.sync_copy(batch_hbm.at[pl.ds(nb + q * ECH, ECH)], bidx)
        for k in range(ECH // L):
            b16 = bidx[pl.ds(k * L, L)]
            m = b16 < G
            bidx[pl.ds(k * L, L)] = jnp.where(m, b16, 0)
            bval[pl.ds(k * L, L)] = jnp.where(m, 1.0, 0.0)
        pltpu.sync_copy(bval, cacc.at[bidx], add=True)

    plsc.subcore_barrier()

    pltpu.sync_copy(dacc.at[pl.ds(s * (NPAD // NS), NPAD // NS)],
                    deg_out.at[c, pl.ds(s * (NPAD // NS), NPAD // NS)])

    @pl.when(s == 0)
    def _():
        pltpu.sync_copy(cacc, cnt_out.at[c])


_BLK = 512
_NB = NPAD // _BLK


def _k0_body(deg_ref, dinv_ref):
    dinv_ref[...] = lax.rsqrt(deg_ref[0] + deg_ref[1] + 1.0)


def _k0(deg_part):
    return pl.pallas_call(
        _k0_body,
        in_specs=[pl.BlockSpec((NC, NPAD, 1), lambda: (0, 0, 0))],
        out_specs=pl.BlockSpec((NPAD, 1), lambda: (0, 0)),
        out_shape=jax.ShapeDtypeStruct((NPAD, 1), jnp.float32),
    )(deg_part)


def _k1_body(dinv_ref, x_ref, w1_ref, h1p_ref):
    h1p_ref[...] = dinv_ref[...] * jnp.dot(x_ref[...], w1_ref[...],
                                           preferred_element_type=jnp.float32)


def _k1(dinv, x, W1):
    return pl.pallas_call(
        _k1_body,
        grid=(_NB,),
        in_specs=[
            pl.BlockSpec((_BLK, 1), lambda b: (b, 0)),
            pl.BlockSpec((_BLK, D), lambda b: (b, 0)),
            pl.BlockSpec((D, D), lambda b: (0, 0)),
        ],
        out_specs=pl.BlockSpec((_BLK, D), lambda b: (b, 0)),
        out_shape=jax.ShapeDtypeStruct((NPAD, D), jnp.float32),
    )(dinv, x, W1)


_RB = 3
_IR = _RB + 1


@functools.partial(
    pl.kernel,
    out_type=jax.ShapeDtypeStruct((NC, NPAD, D), jnp.float32),
    mesh=_mesh,
    scratch_types=(
        pltpu.VMEM_SHARED((NPAD, D), jnp.float32),
        pltpu.VMEM((_RB, ECH, D), jnp.float32),
        pltpu.VMEM((_IR, ECH), jnp.int32),
        pltpu.VMEM((_IR, ECH), jnp.int32),
        pltpu.SemaphoreType.DMA((_RB,)),
        pltpu.SemaphoreType.DMA((_IR,)),
        pltpu.SemaphoreType.DMA((_IR,)),
    ),
)
def _sc_rows(h1p_hbm, src_hbm, dst_hbm, out1_part,
             acc, rowsb, sring, dring, gsem, ssem, dsem):
    c = lax.axis_index("c")
    s = lax.axis_index("s")
    wid = s * NC + c
    rpt = NPAD // NS
    base = wid * NCHUNK

    def load_idx(j):
        r = lax.rem(j, _IR) if not isinstance(j, int) else j % _IR
        e0 = (base + j) * ECH
        pltpu.async_copy(src_hbm.at[pl.ds(e0, ECH)], sring.at[r], ssem.at[r])
        pltpu.async_copy(dst_hbm.at[pl.ds(e0, ECH)], dring.at[r], dsem.at[r])

    def wait_idx(j):
        r = lax.rem(j, _IR) if not isinstance(j, int) else j % _IR
        e0 = (base + j) * ECH
        pltpu.make_async_copy(src_hbm.at[pl.ds(e0, ECH)], sring.at[r],
                              ssem.at[r]).wait()
        pltpu.make_async_copy(dst_hbm.at[pl.ds(e0, ECH)], dring.at[r],
                              dsem.at[r]).wait()

    def issue_gather(j):
        ri = lax.rem(j, _IR) if not isinstance(j, int) else j % _IR
        b = lax.rem(j, _RB) if not isinstance(j, int) else j % _RB
        pltpu.async_copy(h1p_hbm.at[sring.at[ri]], rowsb.at[b], gsem.at[b])

    for j0 in range(_RB):
        load_idx(j0)

    def zr_body(i, _):
        for k in range(D // L):
            rowsb[0, i, pl.ds(k * L, L)] = jnp.zeros((L,), jnp.float32)
        return 0

    lax.fori_loop(0, ECH, zr_body, 0)
    for q in range(rpt // ECH):
        pltpu.sync_copy(rowsb.at[0], acc.at[pl.ds(s * rpt + q * ECH, ECH)])

    plsc.subcore_barrier()

    for j0 in range(_RB - 1):
        wait_idx(j0)
        issue_gather(j0)

    def inner(j, _):
        jn = j + _RB - 1

        @pl.when(jn < NCHUNK)
        def _():
            wait_idx(jn)
            issue_gather(jn)

            @pl.when(jn + 1 < NCHUNK)
            def _():
                load_idx(jn + 1)

        b = lax.rem(j, _RB)
        ri = lax.rem(j, _IR)
        pltpu.make_async_copy(h1p_hbm.at[sring.at[ri]], rowsb.at[b],
                              gsem.at[b]).wait()
        pltpu.sync_copy(rowsb.at[b], acc.at[dring.at[ri]], add=True)
        return 0

    lax.fori_loop(0, NCHUNK, inner, 0)

    plsc.subcore_barrier()
    pltpu.sync_copy(acc.at[pl.ds(s * rpt, rpt)],
                    out1_part.at[c, pl.ds(s * rpt, rpt)])


_NR = NPAD // 128


@functools.partial(
    pl.kernel,
    out_type=jax.ShapeDtypeStruct((NC, CN), jnp.float32),
    mesh=_mesh,
    scratch_types=(
        pltpu.VMEM_SHARED((CN,), jnp.float32),
        pltpu.VMEM((NCHUNK, ECH), jnp.int32),
        pltpu.VMEM((NCHUNK, ECH), jnp.int32),
        pltpu.VMEM((ECH,), jnp.int32),
        pltpu.VMEM((ECH,), jnp.float32),
        pltpu.VMEM((ECH,), jnp.int32),
        pltpu.VMEM((ECH,), jnp.float32),
        pltpu.VMEM((NPW,), jnp.int32),
        pltpu.VMEM((NPW,), jnp.float32),
        pltpu.VMEM((NPAD,), jnp.float32),
        pltpu.SemaphoreType.DMA((4,)),
    ),
)
def _sc_cmat(src_hbm, dst_hbm, dinv_hbm, batch_hbm,
             c_part, cacc, sblk, dblk, cidx, cval, gbv, wbv,
             bsl, dsl, zbuf, isem):
    c = lax.axis_index("c")
    s = lax.axis_index("s")
    wid = s * NC + c
    cpt = CN // NS

    scp = pltpu.async_copy(src_hbm.at[wid], sblk, isem.at[0])
    dcp = pltpu.async_copy(dst_hbm.at[wid], dblk, isem.at[1])

    _zero_1d(zbuf, NPAD)
    for q in range(cpt // NPAD):
        pltpu.sync_copy(zbuf, cacc.at[pl.ds(s * cpt + q * NPAD, NPAD)])

    scp.wait()
    dcp.wait()
    plsc.subcore_barrier()

    def chunk(j, _):
        pltpu.sync_copy(batch_hbm.at[dblk.at[j]], gbv)
        pltpu.sync_copy(dinv_hbm.at[dblk.at[j]], wbv)
        for k in range(ECH // L):
            sl = pl.ds(k * L, L)
            cidx[sl] = gbv[sl] * NPAD + sblk[j, sl]
            cval[sl] = wbv[sl]
        pltpu.sync_copy(cval, cacc.at[cidx], add=True)
        return 0

    lax.fori_loop(0, NCHUNK, chunk, 0)

    nb = wid * NPW
    pltpu.sync_copy(batch_hbm.at[pl.ds(nb, NPW)], bsl)
    pltpu.sync_copy(dinv_hbm.at[pl.ds(nb, NPW)], dsl)
    for q in range(NPW // ECH):
        for k in range(ECH // L):
            off = q * ECH + k * L
            b16 = bsl[pl.ds(off, L)]
            w16 = dsl[pl.ds(off, L)]
            i16 = lax.iota(jnp.int32, L) + (nb + off)
            m = b16 < G
            cidx[pl.ds(k * L, L)] = jnp.where(m, b16 * NPAD + i16, 0)
            cval[pl.ds(k * L, L)] = jnp.where(m, w16, 0.0)
        pltpu.sync_copy(cval, cacc.at[cidx], add=True)

    plsc.subcore_barrier()
    pltpu.sync_copy(cacc.at[pl.ds(s * cpt, cpt)],
                    c_part.at[c, pl.ds(s * cpt, cpt)])


def _k2_body(o_ref, h1p_ref, dinv_ref, w2_ref, b1_ref, c_ref, cnt_ref, b2_ref,
             out_ref, accs_ref):
    b = pl.program_id(0)
    t = (o_ref[0] + o_ref[1] + h1p_ref[...]) * dinv_ref[...] + b1_ref[...]
    a = jnp.where(t > 0, t, 0.01 * t)
    h2p = jnp.dot(a, w2_ref[...], preferred_element_type=jnp.float32)
    h2p = h2p * dinv_ref[...]
    cblk = c_ref[0] + c_ref[1]
    p = jnp.dot(cblk, h2p, preferred_element_type=jnp.float32)

    @pl.when(b == 0)
    def _():
        accs_ref[...] = jnp.zeros_like(accs_ref)

    accs_ref[...] += p

    @pl.when(b == _NB - 1)
    def _():
        cnt = cnt_ref[0] + cnt_ref[1]
        out_ref[...] = (accs_ref[...] / jnp.maximum(cnt, 1.0)
                        + b2_ref[...] * (cnt >= 1.0))


def _k2(out1_part, h1p, dinv, W2, b1, c_part, cnt_part, b2):
    return pl.pallas_call(
        _k2_body,
        grid=(_NB,),
        in_specs=[
            pl.BlockSpec((NC, _BLK, D), lambda b: (0, b, 0)),
            pl.BlockSpec((_BLK, D), lambda b: (b, 0)),
            pl.BlockSpec((_BLK, 1), lambda b: (b, 0)),
            pl.BlockSpec((D, D), lambda b: (0, 0)),
            pl.BlockSpec((1, D), lambda b: (0, 0)),
            pl.BlockSpec((NC, G, _BLK), lambda b: (0, 0, b)),
            pl.BlockSpec((NC, G, 1), lambda b: (0, 0, 0)),
            pl.BlockSpec((1, D), lambda b: (0, 0)),
        ],
        out_specs=pl.BlockSpec((G, D), lambda b: (0, 0)),
        out_shape=jax.ShapeDtypeStruct((G, D), jnp.float32),
        scratch_shapes=[pltpu.VMEM((G, D), jnp.float32)],
    )(out1_part, h1p, dinv, W2, b1, c_part, cnt_part, b2)


def kernel(x, edge_index, batch, W1, b1, W2, b2):
    src = edge_index[0].astype(jnp.int32)
    dst = edge_index[1].astype(jnp.int32)
    batch_p = jnp.concatenate(
        [batch.astype(jnp.int32), jnp.full((NPAD - N,), G, jnp.int32)])
    x_p = jnp.concatenate([x, jnp.zeros((NPAD - N, D), x.dtype)])

    src2 = src.reshape(NW, NCHUNK, ECH)
    dst2 = dst.reshape(NW, NCHUNK, ECH)
    deg_part, cnt_part = _sc_hist(dst2, batch_p)
    dinv = _k0(deg_part.reshape(NC, NPAD, 1))
    c_part = _sc_cmat(src2, dst2, dinv.reshape(NPAD), batch_p)
    h1p = _k1(dinv, x_p, W1)
    out1_part = _sc_rows(h1p, src, dst)
    emb = _k2(out1_part, h1p, dinv, W2, b1.reshape(1, D),
              c_part.reshape(NC, G, NPAD), cnt_part.reshape(NC, G, 1),
              b2.reshape(1, D))
    return emb


# speedup vs baseline: 1.0009x; 1.0009x over previous
import functools

import jax
import jax.numpy as jnp
from jax import lax
from jax.experimental import pallas as pl
from jax.experimental.pallas import tpu as pltpu
from jax.experimental.pallas import tpu_sc as plsc

N = 10000
NPAD = 10240
E = 320000
D = 128
G = 64
NC = 2
NS = 16
L = 16
NW = NC * NS
EPW = E // NW
ECH = 80
NCHUNK = EPW // ECH
NPW = NPAD // NW
CN = G * NPAD

_mesh = plsc.VectorSubcoreMesh(core_axis_name="c", subcore_axis_name="s")


def _zero_1d(ref, n):
    z = jnp.zeros((L,), ref.dtype)

    def body(i, _):
        ref[pl.ds(i * L, L)] = z
        return 0

    lax.fori_loop(0, n // L, body, 0)


@functools.partial(
    pl.kernel,
    out_type=(
        jax.ShapeDtypeStruct((NC, NPAD), jnp.float32),
        jax.ShapeDtypeStruct((NC, G), jnp.float32),
    ),
    mesh=_mesh,
    scratch_types=(
        pltpu.VMEM_SHARED((NPAD,), jnp.float32),
        pltpu.VMEM_SHARED((G,), jnp.float32),
        pltpu.VMEM((NCHUNK, ECH), jnp.int32),
        pltpu.VMEM((ECH,), jnp.float32),
        pltpu.VMEM((ECH,), jnp.int32),
        pltpu.VMEM((ECH,), jnp.float32),
        pltpu.VMEM((NPAD // NS,), jnp.float32),
        pltpu.SemaphoreType.DMA,
    ),
)
def _sc_hist(dst_hbm, batch_hbm, deg_out, cnt_out,
             dacc, cacc, dblk, onev, bidx, bval, zbuf, sem):
    c = lax.axis_index("c")
    s = lax.axis_index("s")
    wid = s * NC + c

    dcp = pltpu.async_copy(dst_hbm.at[wid], dblk, sem)

    _zero_1d(zbuf, NPAD // NS)
    pltpu.sync_copy(zbuf.at[pl.ds(0, NPAD // NS)], dacc.at[pl.ds(s * (NPAD // NS), NPAD // NS)])

    @pl.when(s == 0)
    def _():
        pltpu.sync_copy(zbuf.at[pl.ds(0, G)], cacc)

    one = jnp.ones((L,), jnp.float32)
    for k in range(ECH // L):
        onev[pl.ds(k * L, L)] = one

    dcp.wait()
    plsc.subcore_barrier()

    def deg_body(j, _):
        pltpu.sync_copy(onev, dacc.at[dblk.at[j]], add=True)
        return 0

    lax.fori_loop(0, NCHUNK, deg_body, 0)

    nb = wid * NPW
    for q in range(NPW // ECH):
        pltpu.sync_copy(batch_hbm.at[pl.ds(nb + q * ECH, ECH)], bidx)
        for k in range(ECH // L):
            b16 = bidx[pl.ds(k * L, L)]
            m = b16 < G
            bidx[pl.ds(k * L, L)] = jnp.where(m, b16, 0)
            bval[pl.ds(k * L, L)] = jnp.where(m, 1.0, 0.0)
        pltpu.sync_copy(bval, cacc.at[bidx], add=True)

    plsc.subcore_barrier()

    pltpu.sync_copy(dacc.at[pl.ds(s * (NPAD // NS), NPAD // NS)],
                    deg_out.at[c, pl.ds(s * (NPAD // NS), NPAD // NS)])

    @pl.when(s == 0)
    def _():
        pltpu.sync_copy(cacc, cnt_out.at[c])


_BLK = 512
_NB = NPAD // _BLK


def _k0_body(deg_ref, dinv_ref):
    dinv_ref[...] = lax.rsqrt(deg_ref[0] + deg_ref[1] + 1.0)


def _k0(deg_part):
    return pl.pallas_call(
        _k0_body,
        in_specs=[pl.BlockSpec((NC, NPAD, 1), lambda: (0, 0, 0))],
        out_specs=pl.BlockSpec((NPAD, 1), lambda: (0, 0)),
        out_shape=jax.ShapeDtypeStruct((NPAD, 1), jnp.float32),
    )(deg_part)


def _k1_body(dinv_ref, x_ref, w1_ref, h1p_ref):
    h1p_ref[...] = dinv_ref[...] * jnp.dot(x_ref[...], w1_ref[...],
                                           preferred_element_type=jnp.float32)


def _k1(dinv, x, W1):
    return pl.pallas_call(
        _k1_body,
        grid=(_NB,),
        in_specs=[
            pl.BlockSpec((_BLK, 1), lambda b: (b, 0)),
            pl.BlockSpec((_BLK, D), lambda b: (b, 0)),
            pl.BlockSpec((D, D), lambda b: (0, 0)),
        ],
        out_specs=pl.BlockSpec((_BLK, D), lambda b: (b, 0)),
        out_shape=jax.ShapeDtypeStruct((NPAD, D), jnp.float32),
    )(dinv, x, W1)


_RB = 3
_IR = _RB + 1


@functools.partial(
    pl.kernel,
    out_type=jax.ShapeDtypeStruct((NC, NPAD, D), jnp.float32),
    mesh=_mesh,
    scratch_types=(
        pltpu.VMEM_SHARED((NPAD, D), jnp.float32),
        pltpu.VMEM((_RB, ECH, D), jnp.float32),
        pltpu.VMEM((_IR, ECH), jnp.int32),
        pltpu.VMEM((_IR, ECH), jnp.int32),
        pltpu.SemaphoreType.DMA((_RB,)),
        pltpu.SemaphoreType.DMA((_IR,)),
        pltpu.SemaphoreType.DMA((_IR,)),
    ),
)
def _sc_rows(h1p_hbm, src_hbm, dst_hbm, out1_part,
             acc, rowsb, sring, dring, gsem, ssem, dsem):
    c = lax.axis_index("c")
    s = lax.axis_index("s")
    wid = s * NC + c
    rpt = NPAD // NS
    base = wid * NCHUNK

    def load_idx(j):
        r = lax.rem(j, _IR) if not isinstance(j, int) else j % _IR
        e0 = (base + j) * ECH
        pltpu.async_copy(src_hbm.at[pl.ds(e0, ECH)], sring.at[r], ssem.at[r])
        pltpu.async_copy(dst_hbm.at[pl.ds(e0, ECH)], dring.at[r], dsem.at[r])

    def wait_idx(j):
        r = lax.rem(j, _IR) if not isinstance(j, int) else j % _IR
        e0 = (base + j) * ECH
        pltpu.make_async_copy(src_hbm.at[pl.ds(e0, ECH)], sring.at[r],
                              ssem.at[r]).wait()
        pltpu.make_async_copy(dst_hbm.at[pl.ds(e0, ECH)], dring.at[r],
                              dsem.at[r]).wait()

    def issue_gather(j):
        ri = lax.rem(j, _IR) if not isinstance(j, int) else j % _IR
        b = lax.rem(j, _RB) if not isinstance(j, int) else j % _RB
        pltpu.async_copy(h1p_hbm.at[sring.at[ri]], rowsb.at[b], gsem.at[b])

    for j0 in range(_RB):
        load_idx(j0)

    def zr_body(i, _):
        for k in range(D // L):
            rowsb[0, i, pl.ds(k * L, L)] = jnp.zeros((L,), jnp.float32)
        return 0

    lax.fori_loop(0, ECH, zr_body, 0)
    for q in range(rpt // ECH):
        pltpu.sync_copy(rowsb.at[0], acc.at[pl.ds(s * rpt + q * ECH, ECH)])

    plsc.subcore_barrier()

    for j0 in range(_RB - 1):
        wait_idx(j0)
        issue_gather(j0)

    def inner(j, _):
        jn = j + _RB - 1

        @pl.when(jn < NCHUNK)
        def _():
            wait_idx(jn)
            issue_gather(jn)

            @pl.when(jn + 1 < NCHUNK)
            def _():
                load_idx(jn + 1)

        b = lax.rem(j, _RB)
        ri = lax.rem(j, _IR)
        pltpu.make_async_copy(h1p_hbm.at[sring.at[ri]], rowsb.at[b],
                              gsem.at[b]).wait()
        pltpu.sync_copy(rowsb.at[b], acc.at[dring.at[ri]], add=True)
        return 0

    lax.fori_loop(0, NCHUNK, inner, 0)

    plsc.subcore_barrier()
    pltpu.sync_copy(acc.at[pl.ds(s * rpt, rpt)],
                    out1_part.at[c, pl.ds(s * rpt, rpt)])


_NR = NPAD // 128


@functools.partial(
    pl.kernel,
    out_type=jax.ShapeDtypeStruct((NC, CN), jnp.float32),
    mesh=_mesh,
    scratch_types=(
        pltpu.VMEM_SHARED((CN,), jnp.float32),
        pltpu.VMEM((NCHUNK, ECH), jnp.int32),
        pltpu.VMEM((NCHUNK, ECH), jnp.int32),
        pltpu.VMEM((ECH,), jnp.int32),
        pltpu.VMEM((ECH,), jnp.float32),
        pltpu.VMEM((ECH,), jnp.int32),
        pltpu.VMEM((ECH,), jnp.float32),
        pltpu.VMEM((NPW,), jnp.int32),
        pltpu.VMEM((NPW,), jnp.float32),
        pltpu.VMEM((NPAD,), jnp.float32),
        pltpu.SemaphoreType.DMA((4,)),
    ),
)
def _sc_cmat(src_hbm, dst_hbm, dinv_hbm, batch_hbm,
             c_part, cacc, sblk, dblk, cidx, cval, gbv, wbv,
             bsl, dsl, zbuf, isem):
    c = lax.axis_index("c")
    s = lax.axis_index("s")
    wid = s * NC + c
    cpt = CN // NS

    scp = pltpu.async_copy(src_hbm.at[wid], sblk, isem.at[0])
    dcp = pltpu.async_copy(dst_hbm.at[wid], dblk, isem.at[1])

    _zero_1d(zbuf, NPAD)
    for q in range(cpt // NPAD):
        pltpu.sync_copy(zbuf, cacc.at[pl.ds(s * cpt + q * NPAD, NPAD)])

    scp.wait()
    dcp.wait()
    plsc.subcore_barrier()

    def chunk(j, _):
        pltpu.sync_copy(batch_hbm.at[dblk.at[j]], gbv)
        pltpu.sync_copy(dinv_hbm.at[dblk.at[j]], wbv)
        for k in range(ECH // L):
            sl = pl.ds(k * L, L)
            cidx[sl] = gbv[sl] * NPAD + sblk[j, sl]
            cval[sl] = wbv[sl]
        pltpu.sync_copy(cval, cacc.at[cidx], add=True)
        return 0

    lax.fori_loop(0, NCHUNK, chunk, 0)

    nb = wid * NPW
    pltpu.sync_copy(batch_hbm.at[pl.ds(nb, NPW)], bsl)
    pltpu.sync_copy(dinv_hbm.at[pl.ds(nb, NPW)], dsl)
    for q in range(NPW // ECH):
        for k in range(ECH // L):
            off = q * ECH + k * L
            b16 = bsl[pl.ds(off, L)]
            w16 = dsl[pl.ds(off, L)]
            i16 = lax.iota(jnp.int32, L) + (nb + off)
            m = b16 < G
            cidx[pl.ds(k * L, L)] = jnp.where(m, b16 * NPAD + i16, 0)
            cval[pl.ds(k * L, L)] = jnp.where(m, w16, 0.0)
        pltpu.sync_copy(cval, cacc.at[cidx], add=True)

    plsc.subcore_barrier()
    pltpu.sync_copy(cacc.at[pl.ds(s * cpt, cpt)],
                    c_part.at[c, pl.ds(s * cpt, cpt)])


def _k2_body(o_ref, h1p_ref, dinv_ref, w2_ref, b1_ref, c_ref, cnt_ref, b2_ref,
             out_ref, accs_ref):
    b = pl.program_id(0)
    t = (o_ref[0] + o_ref[1] + h1p_ref[...]) * dinv_ref[...] + b1_ref[...]
    a = jnp.where(t > 0, t, 0.01 * t)
    h2p = jnp.dot(a, w2_ref[...], preferred_element_type=jnp.float32)
    h2p = h2p * dinv_ref[...]
    cblk = c_ref[0] + c_ref[1]
    p = jnp.dot(cblk, h2p, preferred_element_type=jnp.float32)

    @pl.when(b == 0)
    def _():
        accs_ref[...] = jnp.zeros_like(accs_ref)

    accs_ref[...] += p

    @pl.when(b == _NB - 1)
    def _():
        cnt = cnt_ref[0] + cnt_ref[1]
        out_ref[...] = (accs_ref[...] / jnp.maximum(cnt, 1.0)
                        + b2_ref[...] * (cnt >= 1.0))


def _k2(out1_part, h1p, dinv, W2, b1, c_part, cnt_part, b2):
    return pl.pallas_call(
        _k2_body,
        grid=(_NB,),
        in_specs=[
            pl.BlockSpec((NC, _BLK, D), lambda b: (0, b, 0)),
            pl.BlockSpec((_BLK, D), lambda b: (b, 0)),
            pl.BlockSpec((_BLK, 1), lambda b: (b, 0)),
            pl.BlockSpec((D, D), lambda b: (0, 0)),
            pl.BlockSpec((1, D), lambda b: (0, 0)),
            pl.BlockSpec((NC, G, _BLK), lambda b: (0, 0, b)),
            pl.BlockSpec((NC, G, 1), lambda b: (0, 0, 0)),
            pl.BlockSpec((1, D), lambda b: (0, 0)),
        ],
        out_specs=pl.BlockSpec((G, D), lambda b: (0, 0)),
        out_shape=jax.ShapeDtypeStruct((G, D), jnp.float32),
        scratch_shapes=[pltpu.VMEM((G, D), jnp.float32)],
    )(out1_part, h1p, dinv, W2, b1, c_part, cnt_part, b2)


def kernel(x, edge_index, batch, W1, b1, W2, b2):
    src = edge_index[0].astype(jnp.int32)
    dst = edge_index[1].astype(jnp.int32)
    batch_p = jnp.concatenate(
        [batch.astype(jnp.int32), jnp.full((NPAD - N,), G, jnp.int32)])
    x_p = jnp.concatenate([x, jnp.zeros((NPAD - N, D), x.dtype)])

    src2 = src.reshape(NW, NCHUNK, ECH)
    dst2 = dst.reshape(NW, NCHUNK, ECH)
    deg_part, cnt_part = _sc_hist(dst2, batch_p)
    dinv = _k0(deg_part.reshape(NC, NPAD, 1))
    h1p = _k1(dinv, x_p, W1)
    out1_part = _sc_rows(h1p, src, dst)
    c_part = _sc_cmat(src2, dst2, dinv.reshape(NPAD), batch_p)
    emb = _k2(out1_part, h1p, dinv, W2, b1.reshape(1, D),
              c_part.reshape(NC, G, NPAD), cnt_part.reshape(NC, G, 1),
              b2.reshape(1, D))
    return emb


# speedup vs baseline: 1.5074x; 1.5061x over previous
import functools

import jax
import jax.numpy as jnp
from jax import lax
from jax.experimental import pallas as pl
from jax.experimental.pallas import tpu as pltpu
from jax.experimental.pallas import tpu_sc as plsc

N = 10000
NPAD = 10240
E = 320000
D = 128
G = 64
NC = 2
NS = 16
L = 16
NW = NC * NS
EPW = E // NW
ECH = 80
NCHUNK = EPW // ECH
NPW = NPAD // NW
CN = G * NPAD

_mesh = plsc.VectorSubcoreMesh(core_axis_name="c", subcore_axis_name="s")


def _zero_1d(ref, n):
    z = jnp.zeros((L,), ref.dtype)

    def body(i, _):
        ref[pl.ds(i * L, L)] = z
        return 0

    lax.fori_loop(0, n // L, body, 0)


@functools.partial(
    pl.kernel,
    out_type=(
        jax.ShapeDtypeStruct((NC, NPAD), jnp.float32),
        jax.ShapeDtypeStruct((NC, G), jnp.float32),
    ),
    mesh=_mesh,
    scratch_types=(
        pltpu.VMEM_SHARED((NPAD,), jnp.float32),
        pltpu.VMEM_SHARED((G,), jnp.float32),
        pltpu.VMEM((NCHUNK, ECH), jnp.int32),
        pltpu.VMEM((ECH,), jnp.float32),
        pltpu.VMEM((ECH,), jnp.int32),
        pltpu.VMEM((ECH,), jnp.float32),
        pltpu.VMEM((NPAD // NS,), jnp.float32),
        pltpu.SemaphoreType.DMA,
    ),
)
def _sc_hist(dst_hbm, batch_hbm, deg_out, cnt_out,
             dacc, cacc, dblk, onev, bidx, bval, zbuf, sem):
    c = lax.axis_index("c")
    s = lax.axis_index("s")
    wid = s * NC + c

    dcp = pltpu.async_copy(dst_hbm.at[wid], dblk, sem)

    _zero_1d(zbuf, NPAD // NS)
    pltpu.sync_copy(zbuf.at[pl.ds(0, NPAD // NS)], dacc.at[pl.ds(s * (NPAD // NS), NPAD // NS)])

    @pl.when(s == 0)
    def _():
        pltpu.sync_copy(zbuf.at[pl.ds(0, G)], cacc)

    one = jnp.ones((L,), jnp.float32)
    for k in range(ECH // L):
        onev[pl.ds(k * L, L)] = one

    dcp.wait()
    plsc.subcore_barrier()

    def deg_body(j, _):
        pltpu.sync_copy(onev, dacc.at[dblk.at[j]], add=True)
        return 0

    lax.fori_loop(0, NCHUNK, deg_body, 0)

    nb = wid * NPW
    for q in range(NPW // ECH):
        pltpu.sync_copy(batch_hbm.at[pl.ds(nb + q * ECH, ECH)], bidx)
        for k in range(ECH // L):
            b16 = bidx[pl.ds(k * L, L)]
            m = b16 < G
            bidx[pl.ds(k * L, L)] = jnp.where(m, b16, 0)
            bval[pl.ds(k * L, L)] = jnp.where(m, 1.0, 0.0)
        pltpu.sync_copy(bval, cacc.at[bidx], add=True)

    plsc.subcore_barrier()

    pltpu.sync_copy(dacc.at[pl.ds(s * (NPAD // NS), NPAD // NS)],
                    deg_out.at[c, pl.ds(s * (NPAD // NS), NPAD // NS)])

    @pl.when(s == 0)
    def _():
        pltpu.sync_copy(cacc, cnt_out.at[c])


_BLK = 512
_NB = NPAD // _BLK


def _k0_body(deg_ref, dinv_ref):
    dinv_ref[...] = lax.rsqrt(deg_ref[0] + deg_ref[1] + 1.0)


def _k0(deg_part):
    return pl.pallas_call(
        _k0_body,
        in_specs=[pl.BlockSpec((NC, NPAD, 1), lambda: (0, 0, 0))],
        out_specs=pl.BlockSpec((NPAD, 1), lambda: (0, 0)),
        out_shape=jax.ShapeDtypeStruct((NPAD, 1), jnp.float32),
    )(deg_part)


def _k1_body(dinv_ref, x_ref, w1_ref, h1p_ref):
    h1p_ref[...] = dinv_ref[...] * jnp.dot(x_ref[...], w1_ref[...],
                                           preferred_element_type=jnp.float32)


def _k1(dinv, x, W1):
    return pl.pallas_call(
        _k1_body,
        grid=(_NB,),
        in_specs=[
            pl.BlockSpec((_BLK, 1), lambda b: (b, 0)),
            pl.BlockSpec((_BLK, D), lambda b: (b, 0)),
            pl.BlockSpec((D, D), lambda b: (0, 0)),
        ],
        out_specs=pl.BlockSpec((_BLK, D), lambda b: (b, 0)),
        out_shape=jax.ShapeDtypeStruct((NPAD, D), jnp.float32),
    )(dinv, x, W1)


_RB = 3
_IR = _RB + 1


@functools.partial(
    pl.kernel,
    out_type=jax.ShapeDtypeStruct((NC, NPAD, D), jnp.float32),
    mesh=_mesh,
    scratch_types=(
        pltpu.VMEM_SHARED((NPAD, D), jnp.float32),
        pltpu.VMEM((_RB, ECH, D), jnp.float32),
        pltpu.VMEM((_IR, ECH), jnp.int32),
        pltpu.VMEM((_IR, ECH), jnp.int32),
        pltpu.SemaphoreType.DMA((_RB,)),
        pltpu.SemaphoreType.DMA((_IR,)),
        pltpu.SemaphoreType.DMA((_IR,)),
    ),
)
def _sc_rows(h1p_hbm, src_hbm, dst_hbm, out1_part,
             acc, rowsb, sring, dring, gsem, ssem, dsem):
    c = lax.axis_index("c")
    s = lax.axis_index("s")
    wid = s * NC + c
    rpt = NPAD // NS
    base = wid * NCHUNK

    def load_idx(j):
        r = lax.rem(j, _IR) if not isinstance(j, int) else j % _IR
        e0 = (base + j) * ECH
        pltpu.async_copy(src_hbm.at[pl.ds(e0, ECH)], sring.at[r], ssem.at[r])
        pltpu.async_copy(dst_hbm.at[pl.ds(e0, ECH)], dring.at[r], dsem.at[r])

    def wait_idx(j):
        r = lax.rem(j, _IR) if not isinstance(j, int) else j % _IR
        e0 = (base + j) * ECH
        pltpu.make_async_copy(src_hbm.at[pl.ds(e0, ECH)], sring.at[r],
                              ssem.at[r]).wait()
        pltpu.make_async_copy(dst_hbm.at[pl.ds(e0, ECH)], dring.at[r],
                              dsem.at[r]).wait()

    def issue_gather(j):
        ri = lax.rem(j, _IR) if not isinstance(j, int) else j % _IR
        b = lax.rem(j, _RB) if not isinstance(j, int) else j % _RB
        pltpu.async_copy(h1p_hbm.at[sring.at[ri]], rowsb.at[b], gsem.at[b])

    for j0 in range(_RB):
        load_idx(j0)

    def zr_body(i, _):
        for k in range(D // L):
            rowsb[0, i, pl.ds(k * L, L)] = jnp.zeros((L,), jnp.float32)
        return 0

    lax.fori_loop(0, ECH, zr_body, 0)
    for q in range(rpt // ECH):
        pltpu.sync_copy(rowsb.at[0], acc.at[pl.ds(s * rpt + q * ECH, ECH)])

    plsc.subcore_barrier()

    for j0 in range(_RB - 1):
        wait_idx(j0)
        issue_gather(j0)

    def inner(j, _):
        jn = j + _RB - 1

        @pl.when(jn < NCHUNK)
        def _():
            wait_idx(jn)
            issue_gather(jn)

            @pl.when(jn + 1 < NCHUNK)
            def _():
                load_idx(jn + 1)

        b = lax.rem(j, _RB)
        ri = lax.rem(j, _IR)
        pltpu.make_async_copy(h1p_hbm.at[sring.at[ri]], rowsb.at[b],
                              gsem.at[b]).wait()
        pltpu.sync_copy(rowsb.at[b], acc.at[dring.at[ri]], add=True)
        return 0

    lax.fori_loop(0, NCHUNK, inner, 0)

    plsc.subcore_barrier()
    pltpu.sync_copy(acc.at[pl.ds(s * rpt, rpt)],
                    out1_part.at[c, pl.ds(s * rpt, rpt)])


_NR = NPAD // 128


@functools.partial(
    pl.kernel,
    out_type=jax.ShapeDtypeStruct((NC, CN), jnp.float32),
    mesh=_mesh,
    scratch_types=(
        pltpu.VMEM_SHARED((CN,), jnp.float32),
        pltpu.VMEM((NCHUNK, ECH), jnp.int32),
        pltpu.VMEM((NCHUNK, ECH), jnp.int32),
        pltpu.VMEM((ECH,), jnp.int32),
        pltpu.VMEM((ECH,), jnp.float32),
        pltpu.VMEM((2, ECH), jnp.int32),
        pltpu.VMEM((2, ECH), jnp.float32),
        pltpu.VMEM((NPW,), jnp.int32),
        pltpu.VMEM((NPW,), jnp.float32),
        pltpu.VMEM((NPAD,), jnp.float32),
        pltpu.SemaphoreType.DMA((4,)),
        pltpu.SemaphoreType.DMA((2,)),
        pltpu.SemaphoreType.DMA((2,)),
    ),
)
def _sc_cmat(src_hbm, dst_hbm, dinv_hbm, batch_hbm,
             c_part, cacc, sblk, dblk, cidx, cval, gbv, wbv,
             bsl, dsl, zbuf, isem, gsem, wsem):
    c = lax.axis_index("c")
    s = lax.axis_index("s")
    wid = s * NC + c
    cpt = CN // NS

    scp = pltpu.async_copy(src_hbm.at[wid], sblk, isem.at[0])
    dcp = pltpu.async_copy(dst_hbm.at[wid], dblk, isem.at[1])

    _zero_1d(zbuf, NPAD)
    for q in range(cpt // NPAD):
        pltpu.sync_copy(zbuf, cacc.at[pl.ds(s * cpt + q * NPAD, NPAD)])

    scp.wait()
    dcp.wait()
    plsc.subcore_barrier()

    def issue(j, r):
        pltpu.async_copy(batch_hbm.at[dblk.at[j]], gbv.at[r], gsem.at[r])
        pltpu.async_copy(dinv_hbm.at[dblk.at[j]], wbv.at[r], wsem.at[r])

    issue(0, 0)

    def chunk(j, _):
        r = lax.rem(j, 2)

        @pl.when(j + 1 < NCHUNK)
        def _():
            issue(j + 1, lax.rem(j + 1, 2))

        pltpu.make_async_copy(batch_hbm.at[dblk.at[j]], gbv.at[r],
                              gsem.at[r]).wait()
        pltpu.make_async_copy(dinv_hbm.at[dblk.at[j]], wbv.at[r],
                              wsem.at[r]).wait()
        for k in range(ECH // L):
            sl = pl.ds(k * L, L)
            cidx[sl] = gbv[r, sl] * NPAD + sblk[j, sl]
            cval[sl] = wbv[r, sl]
        pltpu.sync_copy(cval, cacc.at[cidx], add=True)
        return 0

    lax.fori_loop(0, NCHUNK, chunk, 0)

    nb = wid * NPW
    pltpu.sync_copy(batch_hbm.at[pl.ds(nb, NPW)], bsl)
    pltpu.sync_copy(dinv_hbm.at[pl.ds(nb, NPW)], dsl)
    for q in range(NPW // ECH):
        for k in range(ECH // L):
            off = q * ECH + k * L
            b16 = bsl[pl.ds(off, L)]
            w16 = dsl[pl.ds(off, L)]
            i16 = lax.iota(jnp.int32, L) + (nb + off)
            m = b16 < G
            cidx[pl.ds(k * L, L)] = jnp.where(m, b16 * NPAD + i16, 0)
            cval[pl.ds(k * L, L)] = jnp.where(m, w16, 0.0)
        pltpu.sync_copy(cval, cacc.at[cidx], add=True)

    plsc.subcore_barrier()
    pltpu.sync_copy(cacc.at[pl.ds(s * cpt, cpt)],
                    c_part.at[c, pl.ds(s * cpt, cpt)])


def _k2_body(o_ref, h1p_ref, dinv_ref, w2_ref, b1_ref, c_ref, cnt_ref, b2_ref,
             out_ref, accs_ref):
    b = pl.program_id(0)
    t = (o_ref[0] + o_ref[1] + h1p_ref[...]) * dinv_ref[...] + b1_ref[...]
    a = jnp.where(t > 0, t, 0.01 * t)
    h2p = jnp.dot(a, w2_ref[...], preferred_element_type=jnp.float32)
    h2p = h2p * dinv_ref[...]
    cblk = c_ref[0] + c_ref[1]
    p = jnp.dot(cblk, h2p, preferred_element_type=jnp.float32)

    @pl.when(b == 0)
    def _():
        accs_ref[...] = jnp.zeros_like(accs_ref)

    accs_ref[...] += p

    @pl.when(b == _NB - 1)
    def _():
        cnt = cnt_ref[0] + cnt_ref[1]
        out_ref[...] = (accs_ref[...] / jnp.maximum(cnt, 1.0)
                        + b2_ref[...] * (cnt >= 1.0))


def _k2(out1_part, h1p, dinv, W2, b1, c_part, cnt_part, b2):
    return pl.pallas_call(
        _k2_body,
        grid=(_NB,),
        in_specs=[
            pl.BlockSpec((NC, _BLK, D), lambda b: (0, b, 0)),
            pl.BlockSpec((_BLK, D), lambda b: (b, 0)),
            pl.BlockSpec((_BLK, 1), lambda b: (b, 0)),
            pl.BlockSpec((D, D), lambda b: (0, 0)),
            pl.BlockSpec((1, D), lambda b: (0, 0)),
            pl.BlockSpec((NC, G, _BLK), lambda b: (0, 0, b)),
            pl.BlockSpec((NC, G, 1), lambda b: (0, 0, 0)),
            pl.BlockSpec((1, D), lambda b: (0, 0)),
        ],
        out_specs=pl.BlockSpec((G, D), lambda b: (0, 0)),
        out_shape=jax.ShapeDtypeStruct((G, D), jnp.float32),
        scratch_shapes=[pltpu.VMEM((G, D), jnp.float32)],
    )(out1_part, h1p, dinv, W2, b1, c_part, cnt_part, b2)


def kernel(x, edge_index, batch, W1, b1, W2, b2):
    src = edge_index[0].astype(jnp.int32)
    dst = edge_index[1].astype(jnp.int32)
    batch_p = jnp.concatenate(
        [batch.astype(jnp.int32), jnp.full((NPAD - N,), G, jnp.int32)])
    x_p = jnp.concatenate([x, jnp.zeros((NPAD - N, D), x.dtype)])

    src2 = src.reshape(NW, NCHUNK, ECH)
    dst2 = dst.reshape(NW, NCHUNK, ECH)
    deg_part, cnt_part = _sc_hist(dst2, batch_p)
    dinv = _k0(deg_part.reshape(NC, NPAD, 1))
    h1p = _k1(dinv, x_p, W1)
    out1_part = _sc_rows(h1p, src, dst)
    c_part = _sc_cmat(src2, dst2, dinv.reshape(NPAD), batch_p)
    emb = _k2(out1_part, h1p, dinv, W2, b1.reshape(1, D),
              c_part.reshape(NC, G, NPAD), cnt_part.reshape(NC, G, 1),
              b2.reshape(1, D))
    return emb
